# baseline (speedup 1.0000x reference)
import functools
import math

import jax
import jax.numpy as jnp
import numpy as np
from jax import lax
from jax.experimental import pallas as pl
from jax.experimental.pallas import tpu as pltpu
from jax.experimental.pallas import tpu_sc as plsc

VOCAB = 1000000
D = 64
MAXTOK = 20
B = 1024
S = 50
EPS = 1e-5

N = B * S
NC, NS = 2, 16
NW = NC * NS
EV_PER_TILE = N // NW
CHUNK = 32
NCHUNK = EV_PER_TILE // CHUNK
NPAIR = NCHUNK // 2
L = 16
NBLK = CHUNK // L
NG = 2
TG = MAXTOK // NG
LINE = 2 * D
GBYTES = TG * CHUNK * LINE * 4


def _pe_mean() -> np.ndarray:
    position = np.arange(MAXTOK, dtype=np.float64)[:, None]
    div_term = np.exp(
        np.arange(0, D, 2, dtype=np.float64) * (-math.log(10000.0) / D))
    pe = np.zeros((MAXTOK, D), dtype=np.float64)
    pe[:, 0::2] = np.sin(position * div_term)
    pe[:, 1::2] = np.cos(position * div_term)
    return pe.mean(axis=0).astype(np.float32)


def _rsqrt(x):
    i = plsc.bitcast(x, jnp.int32)
    i = jnp.int32(0x5F3759DF) - lax.shift_right_logical(i, 1)
    y = plsc.bitcast(i, jnp.float32)
    half = x * 0.5
    for _ in range(3):
        y = y * (1.5 - half * y * y)
    return y


NBFULL = VOCAB // 128
NPAIR_TR = 123


def _tr_body(tabt_h, tail_h, out_h, tin_a, tin_b, tout_a, tout_b,
             tin_p, tout_p, insem, outsem):
    cid = lax.axis_index("c")
    sid = lax.axis_index("s")
    wid = sid * NC + cid
    iota = lax.iota(jnp.int32, L)
    nfull = jnp.where(wid < NBFULL % NW, NBFULL // NW + 1, NBFULL // NW)

    def fire_in(j, tin):
        b = wid + NW * j
        pltpu.async_copy(tabt_h.at[pl.ds(0, D), pl.ds(b * 128, 128)],
                         tin, insem)

    def wait_in():
        pltpu.make_async_copy(tabt_h.at[pl.ds(0, D), pl.ds(0, 128)],
                              tin_a, insem).wait()

    def fire_out(j, tout):
        b = wid + NW * j
        pltpu.async_copy(tout, out_h.at[pl.ds(b * 64, 64)], outsem)

    def wait_out():
        pltpu.make_async_copy(tout_a, out_h.at[pl.ds(0, 64)], outsem).wait()

    def transpose(tin, tout, nlines):
        for ib in range(nlines // L):
            lines16 = iota + ib * L
            lines2 = lines16 * 2

            def kk(k2, jv, lines16=lines16, lines2=lines2, tin=tin,
                   tout=tout):
                for u in range(8):
                    h = lax.shift_right_logical(jv, 6)
                    dd = jnp.bitwise_and(jv, 63)
                    val = plsc.load_gather(tin, [dd, lines2 + h])
                    plsc.store_scatter(tout, [lines16, jv], val)
                    jv = jnp.bitwise_and(jv + 1, 127)
                return jv

            lax.fori_loop(0, 16, kk, iota)

    fire_in(0, tin_a)
    fire_in(1, tin_b)

    def pair_body(k, _):
        for sub, tin, tout in ((0, tin_a, tout_a), (1, tin_b, tout_b)):
            j = 2 * k + sub

            @pl.when(j < nfull)
            def _(j=j, tin=tin, tout=tout, k=k):
                wait_in()

                @pl.when(k >= 1)
                def _():
                    wait_out()

                transpose(tin, tout, 64)

                @pl.when(j + 2 < nfull)
                def _(j=j, tin=tin):
                    fire_in(j + 2, tin)

                fire_out(j, tout)
        return 0

    lax.fori_loop(0, NPAIR_TR, pair_body, 0)
    wait_out()
    wait_out()

    @pl.when(wid == 0)
    def _():
        pltpu.sync_copy(tail_h, tin_p)
        transpose(tin_p, tout_p, 32)
        pltpu.sync_copy(tout_p, out_h.at[pl.ds(NBFULL * 64, 32)])


def _sc_body(ids_h, tab_h, gam_h, bet_h, pem_h, out_h,
             idx_all, pair_flat, parb_tok, rows, acc, cnt0_v,
             line0_v, gam_v, bet_v, pem_v, sem):
    cid = lax.axis_index("c")
    sid = lax.axis_index("s")
    wid = sid * NC + cid
    base0 = wid * EV_PER_TILE

    pltpu.sync_copy(tab_h.at[0], line0_v)
    pltpu.sync_copy(gam_h, gam_v)
    pltpu.sync_copy(bet_h, bet_v)
    pltpu.sync_copy(pem_h, pem_v)
    pltpu.sync_copy(ids_h.at[pl.ds(base0, EV_PER_TILE)], idx_all)

    iota = lax.iota(jnp.int32, L)
    zerov = jnp.zeros((L,), jnp.float32)
    inv_tok = jnp.float32(1.0 / MAXTOK)
    inv_d = jnp.float32(1.0 / D)

    def build(ci, ps):
        def t_body(t, cnts):
            tt = jnp.full((L,), t, jnp.int32)
            new = []
            for blk in range(NBLK):
                ev = jnp.full((L,), 0, jnp.int32) + (ci * CHUNK + blk * L) \
                    + iota
                v = plsc.load_gather(idx_all, [ev, tt])
                plsc.store_scatter(
                    pair_flat,
                    [t * CHUNK + (blk * L + ps * MAXTOK * CHUNK) + iota],
                    lax.shift_right_logical(v, 1))
                plsc.store_scatter(
                    parb_tok, [tt + ps * MAXTOK, iota + blk * L],
                    lax.shift_left(jnp.bitwise_and(v, 1), 6))
                new.append(cnts[blk] + jnp.where(v == 0, 1.0, 0.0))
            return tuple(new)

        cnts = lax.fori_loop(0, MAXTOK, t_body,
                             tuple(zerov for _ in range(NBLK)))
        for blk in range(NBLK):
            cnt0_v[pl.ds(ps * CHUNK + blk * L, L)] = cnts[blk]

    def fire(g, ps):
        pltpu.async_copy(
            tab_h.at[pair_flat.at[pl.ds((ps * MAXTOK + g * TG) * CHUNK,
                                        TG * CHUNK)]],
            rows.at[pl.ds(g * TG * CHUNK, TG * CHUNK)], sem)

    def wait_one():
        pltpu.make_async_copy(
            tab_h.at[pair_flat.at[pl.ds(0, TG * CHUNK)]],
            rows.at[pl.ds(0, TG * CHUNK)], sem).wait()

    def acc_group(g, ps):
        for blk in range(NBLK):
            rows16 = iota + blk * L
            parbs = [parb_tok[ps * MAXTOK + g * TG + t, pl.ds(blk * L, L)]
                     for t in range(TG)]
            rowvs = [jnp.full((L,), (g * TG + t) * CHUNK, jnp.int32)
                     + rows16 for t in range(TG)]

            def k_body(k, dvec, g=g, blk=blk, rows16=rows16,
                       parbs=parbs, rowvs=rowvs):
                for _ in range(2):
                    s = zerov
                    for t in range(TG):
                        s = s + plsc.load_gather(
                            rows, [rowvs[t], parbs[t] + dvec])
                    if g == 0:
                        plsc.store_scatter(acc, [rows16, dvec], s)
                    else:
                        cur = plsc.load_gather(acc, [rows16, dvec])
                        plsc.store_scatter(acc, [rows16, dvec], cur + s)
                    dvec = jnp.bitwise_and(dvec + 1, D - 1)
                return dvec

            lax.fori_loop(0, D // 2, k_body, iota)

    def norm_out(ci, ps):
        cblk = [cnt0_v[pl.ds(ps * CHUNK + blk * L, L)]
                for blk in range(NBLK)]

        def p1_body(k, carry):
            dvec = carry[-1]
            sums = list(carry[:-1])
            for _ in range(2):
                r0 = plsc.load_gather(line0_v, [dvec])
                pm = plsc.load_gather(pem_v, [dvec])
                for blk in range(NBLK):
                    rows16 = iota + blk * L
                    c = plsc.load_gather(acc, [rows16, dvec])
                    v = (c - cblk[blk] * r0) * inv_tok + pm
                    sums[blk] = sums[blk] + v
                    sums[NBLK + blk] = sums[NBLK + blk] + v * v
                dvec = jnp.bitwise_and(dvec + 1, D - 1)
            return tuple(sums) + (dvec,)

        carry = lax.fori_loop(0, D // 2, p1_body,
                              tuple(zerov for _ in range(2 * NBLK)) + (iota,))
        mus, rss = [], []
        for blk in range(NBLK):
            mu = carry[blk] * inv_d
            var = carry[NBLK + blk] * inv_d - mu * mu
            mus.append(mu)
            rss.append(_rsqrt(var + EPS))

        def p2_body(k, dvec):
            for _ in range(2):
                r0 = plsc.load_gather(line0_v, [dvec])
                pm = plsc.load_gather(pem_v, [dvec])
                gg = plsc.load_gather(gam_v, [dvec])
                bb = plsc.load_gather(bet_v, [dvec])
                for blk in range(NBLK):
                    rows16 = iota + blk * L
                    c = plsc.load_gather(acc, [rows16, dvec])
                    v = (c - cblk[blk] * r0) * inv_tok + pm
                    o = (v - mus[blk]) * rss[blk] * gg + bb
                    plsc.store_scatter(acc, [rows16, dvec], o)
                dvec = jnp.bitwise_and(dvec + 1, D - 1)
            return dvec

        lax.fori_loop(0, D // 2, p2_body, iota)
        pltpu.sync_copy(acc, out_h.at[pl.ds(base0 + ci * CHUNK, CHUNK)])

    build(0, 0)
    fire(0, 0)
    fire(1, 0)

    def pair_body(k2, _):
        ci_a = 2 * k2
        wait_one()
        acc_group(0, 0)
        build(ci_a + 1, 1)
        fire(0, 1)
        wait_one()
        acc_group(1, 0)
        fire(1, 1)
        norm_out(ci_a, 0)

        wait_one()
        acc_group(0, 1)

        @pl.when(k2 < NPAIR - 1)
        def _():
            build(ci_a + 2, 0)
            fire(0, 0)

        wait_one()
        acc_group(1, 1)

        @pl.when(k2 < NPAIR - 1)
        def _():
            fire(1, 0)

        norm_out(ci_a + 1, 1)
        return 0

    lax.fori_loop(0, NPAIR, pair_body, 0)


@jax.jit
def kernel(input_ids, token_table, ln_gamma, ln_beta):
    ids = input_ids.reshape(N, MAXTOK)
    pe_mean = jnp.asarray(_pe_mean())

    mesh = plsc.VectorSubcoreMesh(core_axis_name="c", subcore_axis_name="s",
                                  num_cores=NC, num_subcores=NS)
    run_tr = pl.kernel(
        _tr_body,
        out_type=jax.ShapeDtypeStruct((VOCAB // 2, LINE), jnp.float32),
        mesh=mesh,
        compiler_params=pltpu.CompilerParams(needs_layout_passes=False),
        scratch_types=[
            pltpu.VMEM((D, 128), jnp.float32),
            pltpu.VMEM((D, 128), jnp.float32),
            pltpu.VMEM((64, LINE), jnp.float32),
            pltpu.VMEM((64, LINE), jnp.float32),
            pltpu.VMEM((D, 64), jnp.float32),
            pltpu.VMEM((32, LINE), jnp.float32),
            pltpu.SemaphoreType.DMA,
            pltpu.SemaphoreType.DMA,
        ],
    )
    tabt = token_table.T
    tab2 = run_tr(tabt, tabt[:, (NBFULL * 128):])

    run = pl.kernel(
        _sc_body,
        out_type=jax.ShapeDtypeStruct((N, D), jnp.float32),
        mesh=mesh,
        compiler_params=pltpu.CompilerParams(
            needs_layout_passes=False, use_tc_tiling_on_sc=False),
        scratch_types=[
            pltpu.VMEM((EV_PER_TILE, MAXTOK), jnp.int32),
            pltpu.VMEM((2 * MAXTOK * CHUNK,), jnp.int32),
            pltpu.VMEM((2 * MAXTOK, CHUNK), jnp.int32),
            pltpu.VMEM((MAXTOK * CHUNK, LINE), jnp.float32),
            pltpu.VMEM((CHUNK, D), jnp.float32),
            pltpu.VMEM((2 * CHUNK,), jnp.float32),
            pltpu.VMEM((LINE,), jnp.float32),
            pltpu.VMEM((D,), jnp.float32),
            pltpu.VMEM((D,), jnp.float32),
            pltpu.VMEM((D,), jnp.float32),
            pltpu.SemaphoreType.DMA,
        ],
    )
    out = run(ids, tab2, ln_gamma, ln_beta, pe_mean)
    return out.reshape(B, S, D)


# speedup vs baseline: 1.0071x; 1.0071x over previous
import functools
import math

import jax
import jax.numpy as jnp
import numpy as np
from jax import lax
from jax.experimental import pallas as pl
from jax.experimental.pallas import tpu as pltpu
from jax.experimental.pallas import tpu_sc as plsc

VOCAB = 1000000
D = 64
MAXTOK = 20
B = 1024
S = 50
EPS = 1e-5

N = B * S
NC, NS = 2, 16
NW = NC * NS
EV_PER_TILE = N // NW
CHUNK = 32
NCHUNK = EV_PER_TILE // CHUNK
NPAIR = NCHUNK // 2
L = 16
NBLK = CHUNK // L
NG = 2
TG = MAXTOK // NG
LINE = 2 * D
GBYTES = TG * CHUNK * LINE * 4


def _pe_mean() -> np.ndarray:
    position = np.arange(MAXTOK, dtype=np.float64)[:, None]
    div_term = np.exp(
        np.arange(0, D, 2, dtype=np.float64) * (-math.log(10000.0) / D))
    pe = np.zeros((MAXTOK, D), dtype=np.float64)
    pe[:, 0::2] = np.sin(position * div_term)
    pe[:, 1::2] = np.cos(position * div_term)
    return pe.mean(axis=0).astype(np.float32)


def _rsqrt(x):
    i = plsc.bitcast(x, jnp.int32)
    i = jnp.int32(0x5F3759DF) - lax.shift_right_logical(i, 1)
    y = plsc.bitcast(i, jnp.float32)
    half = x * 0.5
    for _ in range(3):
        y = y * (1.5 - half * y * y)
    return y


NBFULL = VOCAB // 128
NPAIR_TR = 123


def _tr_body(tabt_h, tail_h, out_h, tin_a, tin_b, tout_a, tout_b,
             tin_p, tout_p, insem, outsem):
    cid = lax.axis_index("c")
    sid = lax.axis_index("s")
    wid = sid * NC + cid
    iota = lax.iota(jnp.int32, L)
    nfull = jnp.where(wid < NBFULL % NW, NBFULL // NW + 1, NBFULL // NW)

    def fire_in(j, tin):
        b = wid + NW * j
        pltpu.async_copy(tabt_h.at[pl.ds(0, D), pl.ds(b * 128, 128)],
                         tin, insem)

    def wait_in():
        pltpu.make_async_copy(tabt_h.at[pl.ds(0, D), pl.ds(0, 128)],
                              tin_a, insem).wait()

    def fire_out(j, tout):
        b = wid + NW * j
        pltpu.async_copy(tout, out_h.at[pl.ds(b * 64, 64)], outsem)

    def wait_out():
        pltpu.make_async_copy(tout_a, out_h.at[pl.ds(0, 64)], outsem).wait()

    def transpose(tin, tout, nlines):
        for ib in range(nlines // L):
            lines16 = iota + ib * L
            lines2 = lines16 * 2

            def kk(k2, _, lines16=lines16, lines2=lines2, tin=tin,
                   tout=tout):
                for u in range(8):
                    jv = jnp.bitwise_and(
                        jnp.full((L,), k2 * 8 + u, jnp.int32) + iota, 127)
                    h = lax.shift_right_logical(jv, 6)
                    dd = jnp.bitwise_and(jv, 63)
                    val = plsc.load_gather(tin, [dd, lines2 + h])
                    plsc.store_scatter(tout, [lines16, jv], val)
                return 0

            lax.fori_loop(0, 16, kk, 0)

    fire_in(0, tin_a)
    fire_in(1, tin_b)

    def pair_body(k, _):
        for sub, tin, tout in ((0, tin_a, tout_a), (1, tin_b, tout_b)):
            j = 2 * k + sub

            @pl.when(j < nfull)
            def _(j=j, tin=tin, tout=tout, k=k):
                wait_in()

                @pl.when(k >= 1)
                def _():
                    wait_out()

                transpose(tin, tout, 64)

                @pl.when(j + 2 < nfull)
                def _(j=j, tin=tin):
                    fire_in(j + 2, tin)

                fire_out(j, tout)
        return 0

    lax.fori_loop(0, NPAIR_TR, pair_body, 0)
    wait_out()
    wait_out()

    @pl.when(wid == 0)
    def _():
        pltpu.sync_copy(tail_h, tin_p)
        transpose(tin_p, tout_p, 32)
        pltpu.sync_copy(tout_p, out_h.at[pl.ds(NBFULL * 64, 32)])


def _sc_body(ids_h, tab_h, gam_h, bet_h, pem_h, out_h,
             idx_all, pair_flat, parb_tok, rows, acc, cnt0_v,
             line0_v, gam_v, bet_v, pem_v, sem):
    cid = lax.axis_index("c")
    sid = lax.axis_index("s")
    wid = sid * NC + cid
    base0 = wid * EV_PER_TILE

    pltpu.sync_copy(tab_h.at[0], line0_v)
    pltpu.sync_copy(gam_h, gam_v)
    pltpu.sync_copy(bet_h, bet_v)
    pltpu.sync_copy(pem_h, pem_v)
    pltpu.sync_copy(ids_h.at[pl.ds(base0, EV_PER_TILE)], idx_all)

    iota = lax.iota(jnp.int32, L)
    zerov = jnp.zeros((L,), jnp.float32)
    inv_tok = jnp.float32(1.0 / MAXTOK)
    inv_d = jnp.float32(1.0 / D)

    def build(ci, ps):
        def t_body(t, cnts):
            tt = jnp.full((L,), t, jnp.int32)
            new = []
            for blk in range(NBLK):
                ev = jnp.full((L,), 0, jnp.int32) + (ci * CHUNK + blk * L) \
                    + iota
                v = plsc.load_gather(idx_all, [ev, tt])
                plsc.store_scatter(
                    pair_flat,
                    [t * CHUNK + (blk * L + ps * MAXTOK * CHUNK) + iota],
                    lax.shift_right_logical(v, 1))
                plsc.store_scatter(
                    parb_tok, [tt + ps * MAXTOK, iota + blk * L],
                    lax.shift_left(jnp.bitwise_and(v, 1), 6))
                new.append(cnts[blk] + jnp.where(v == 0, 1.0, 0.0))
            return tuple(new)

        cnts = lax.fori_loop(0, MAXTOK, t_body,
                             tuple(zerov for _ in range(NBLK)))
        for blk in range(NBLK):
            cnt0_v[pl.ds(ps * CHUNK + blk * L, L)] = cnts[blk]

    def fire(g, ps):
        pltpu.async_copy(
            tab_h.at[pair_flat.at[pl.ds((ps * MAXTOK + g * TG) * CHUNK,
                                        TG * CHUNK)]],
            rows.at[pl.ds(g * TG * CHUNK, TG * CHUNK)], sem)

    def wait_one():
        pltpu.make_async_copy(
            tab_h.at[pair_flat.at[pl.ds(0, TG * CHUNK)]],
            rows.at[pl.ds(0, TG * CHUNK)], sem).wait()

    def acc_group(g, ps):
        for blk in range(NBLK):
            rows16 = iota + blk * L
            parbs = [parb_tok[ps * MAXTOK + g * TG + t, pl.ds(blk * L, L)]
                     for t in range(TG)]
            rowvs = [jnp.full((L,), (g * TG + t) * CHUNK, jnp.int32)
                     + rows16 for t in range(TG)]

            def k_body(k, _, g=g, blk=blk, rows16=rows16,
                       parbs=parbs, rowvs=rowvs):
                dvec = jnp.bitwise_and(jnp.full((L,), k, jnp.int32)
                                       + iota, D - 1)
                s = zerov
                for t in range(TG):
                    s = s + plsc.load_gather(
                        rows, [rowvs[t], parbs[t] + dvec])
                if g == 0:
                    plsc.store_scatter(acc, [rows16, dvec], s)
                else:
                    cur = plsc.load_gather(acc, [rows16, dvec])
                    plsc.store_scatter(acc, [rows16, dvec], cur + s)
                return 0

            lax.fori_loop(0, D, k_body, 0)

    def norm_out(ci, ps):
        cblk = [cnt0_v[pl.ds(ps * CHUNK + blk * L, L)]
                for blk in range(NBLK)]

        def p1_body(k, carry):
            dvec = jnp.bitwise_and(jnp.full((L,), k, jnp.int32) + iota, D - 1)
            r0 = plsc.load_gather(line0_v, [dvec])
            pm = plsc.load_gather(pem_v, [dvec])
            new1, new2 = [], []
            for blk in range(NBLK):
                rows16 = iota + blk * L
                c = plsc.load_gather(acc, [rows16, dvec])
                v = (c - cblk[blk] * r0) * inv_tok + pm
                new1.append(carry[blk] + v)
                new2.append(carry[NBLK + blk] + v * v)
            return tuple(new1) + tuple(new2)

        carry = lax.fori_loop(0, D, p1_body,
                              tuple(zerov for _ in range(2 * NBLK)))
        mus, rss = [], []
        for blk in range(NBLK):
            mu = carry[blk] * inv_d
            var = carry[NBLK + blk] * inv_d - mu * mu
            mus.append(mu)
            rss.append(_rsqrt(var + EPS))

        def p2_body(k, _):
            dvec = jnp.bitwise_and(jnp.full((L,), k, jnp.int32) + iota, D - 1)
            r0 = plsc.load_gather(line0_v, [dvec])
            pm = plsc.load_gather(pem_v, [dvec])
            gg = plsc.load_gather(gam_v, [dvec])
            bb = plsc.load_gather(bet_v, [dvec])
            for blk in range(NBLK):
                rows16 = iota + blk * L
                c = plsc.load_gather(acc, [rows16, dvec])
                v = (c - cblk[blk] * r0) * inv_tok + pm
                o = (v - mus[blk]) * rss[blk] * gg + bb
                plsc.store_scatter(acc, [rows16, dvec], o)
            return 0

        lax.fori_loop(0, D, p2_body, 0)
        pltpu.sync_copy(acc, out_h.at[pl.ds(base0 + ci * CHUNK, CHUNK)])

    build(0, 0)
    fire(0, 0)
    fire(1, 0)

    def pair_body(k2, _):
        ci_a = 2 * k2
        wait_one()
        acc_group(0, 0)
        build(ci_a + 1, 1)
        fire(0, 1)
        wait_one()
        acc_group(1, 0)
        fire(1, 1)
        norm_out(ci_a, 0)

        wait_one()
        acc_group(0, 1)

        @pl.when(k2 < NPAIR - 1)
        def _():
            build(ci_a + 2, 0)
            fire(0, 0)

        wait_one()
        acc_group(1, 1)

        @pl.when(k2 < NPAIR - 1)
        def _():
            fire(1, 0)

        norm_out(ci_a + 1, 1)
        return 0

    lax.fori_loop(0, NPAIR, pair_body, 0)


@jax.jit
def kernel(input_ids, token_table, ln_gamma, ln_beta):
    ids = input_ids.reshape(N, MAXTOK)
    pe_mean = jnp.asarray(_pe_mean())

    mesh = plsc.VectorSubcoreMesh(core_axis_name="c", subcore_axis_name="s",
                                  num_cores=NC, num_subcores=NS)
    run_tr = pl.kernel(
        _tr_body,
        out_type=jax.ShapeDtypeStruct((VOCAB // 2, LINE), jnp.float32),
        mesh=mesh,
        compiler_params=pltpu.CompilerParams(needs_layout_passes=False),
        scratch_types=[
            pltpu.VMEM((D, 128), jnp.float32),
            pltpu.VMEM((D, 128), jnp.float32),
            pltpu.VMEM((64, LINE), jnp.float32),
            pltpu.VMEM((64, LINE), jnp.float32),
            pltpu.VMEM((D, 64), jnp.float32),
            pltpu.VMEM((32, LINE), jnp.float32),
            pltpu.SemaphoreType.DMA,
            pltpu.SemaphoreType.DMA,
        ],
    )
    tabt = token_table.T
    tab2 = run_tr(tabt, tabt[:, (NBFULL * 128):])

    run = pl.kernel(
        _sc_body,
        out_type=jax.ShapeDtypeStruct((N, D), jnp.float32),
        mesh=mesh,
        compiler_params=pltpu.CompilerParams(
            needs_layout_passes=False, use_tc_tiling_on_sc=False),
        scratch_types=[
            pltpu.VMEM((EV_PER_TILE, MAXTOK), jnp.int32),
            pltpu.VMEM((2 * MAXTOK * CHUNK,), jnp.int32),
            pltpu.VMEM((2 * MAXTOK, CHUNK), jnp.int32),
            pltpu.VMEM((MAXTOK * CHUNK, LINE), jnp.float32),
            pltpu.VMEM((CHUNK, D), jnp.float32),
            pltpu.VMEM((2 * CHUNK,), jnp.float32),
            pltpu.VMEM((LINE,), jnp.float32),
            pltpu.VMEM((D,), jnp.float32),
            pltpu.VMEM((D,), jnp.float32),
            pltpu.VMEM((D,), jnp.float32),
            pltpu.SemaphoreType.DMA,
        ],
    )
    out = run(ids, tab2, ln_gamma, ln_beta, pe_mean)
    return out.reshape(B, S, D)
